# baseline (speedup 1.0000x reference)
import functools

import jax
import jax.numpy as jnp
from jax import lax
from jax.experimental import pallas as pl
from jax.experimental.pallas import tpu as pltpu
from jax.experimental.pallas import tpu_sc as plsc

B = 4096
F = 26
U1 = 64
U2 = 32

NC = 2
NS = 16
NW = NC * NS
RPW = B // NW
SUB = 4
G = RPW // SUB
CH = SUB * F

_PERM = [e for h in range(2) for par in range(2)
         for e in range(h * 32 + par, h * 32 + 32, 2)]


def _sc_embed(idx, w0, w1, table2):
  mesh = plsc.VectorSubcoreMesh(
      core_axis_name="c", subcore_axis_name="s", num_cores=NC, num_subcores=NS)

  @functools.partial(
      pl.kernel,
      mesh=mesh,
      out_type=jax.ShapeDtypeStruct((B, U1), jnp.float32),
      compiler_params=pltpu.CompilerParams(
          use_tc_tiling_on_sc=False, needs_layout_passes=False),
      scratch_types=[
          pltpu.VMEM((G, CH), jnp.int32),
          pltpu.VMEM((RPW, 32), jnp.float32),
          pltpu.VMEM((RPW, 32), jnp.float32),
          pltpu.VMEM((CH, 128), jnp.bfloat16),
          pltpu.VMEM((RPW, U1), jnp.float32),
          pltpu.SemaphoreType.DMA,
      ],
  )
  def k(idx_hbm, w0_hbm, w1_hbm, tab_hbm, out_hbm,
        idx_v, w0_v, w1_v, rows_v, out_v, sem):
    wid = lax.axis_index("s") * NC + lax.axis_index("c")
    base = wid * RPW
    pltpu.sync_copy(idx_hbm.at[wid], idx_v)
    pltpu.sync_copy(w0_hbm.at[pl.ds(base, RPW)], w0_v)
    pltpu.sync_copy(w1_hbm.at[pl.ds(base, RPW)], w1_v)

    def chunk_body(g, carry):
      pltpu.async_copy(tab_hbm.at[idx_v.at[g]], rows_v, sem).wait()

      def row_body(bl, carry2):
        row = g * SUB + bl
        rbase = bl * F
        wv0 = [w0_v[row, pl.ds(h * 16, 16)] for h in range(2)]
        wv1 = [w1_v[row, pl.ds(h * 16, 16)] for h in range(2)]
        acc = [jnp.zeros((16,), jnp.float32) for _ in range(4)]
        for f in range(F):
          a0 = wv0[f // 16][f % 16]
          a1 = wv1[f // 16][f % 16]
          r = rbase + f
          for h in range(2):
            e0, o0 = plsc.unpack(rows_v[r, pl.ds(h * 32, 32)],
                                 format=plsc.PackFormat.INTERLEAVED)
            e1, o1 = plsc.unpack(rows_v[r, pl.ds(64 + h * 32, 32)],
                                 format=plsc.PackFormat.INTERLEAVED)
            acc[2 * h] = acc[2 * h] + a0 * e0 + a1 * e1
            acc[2 * h + 1] = acc[2 * h + 1] + a0 * o0 + a1 * o1
        for j in range(4):
          out_v[row, pl.ds(j * 16, 16)] = acc[j]
        return carry2

      return lax.fori_loop(0, SUB, row_body, carry)

    lax.fori_loop(0, G, chunk_body, 0)
    pltpu.sync_copy(out_v, out_hbm.at[pl.ds(base, RPW)])

  return k(idx, w0, w1, table2)


def _tc_tail(x, b1, k2, b2):
  TB = 512

  def body(x_ref, b1_ref, k2_ref, b2_ref, o_ref):
    xb = jnp.maximum(x_ref[...] + b1_ref[...], 0.0)
    y = jnp.dot(xb, k2_ref[...], preferred_element_type=jnp.float32)
    o_ref[...] = jax.nn.sigmoid(y + b2_ref[...])

  return pl.pallas_call(
      body,
      grid=(B // TB,),
      in_specs=[
          pl.BlockSpec((TB, U1), lambda i: (i, 0)),
          pl.BlockSpec((1, U1), lambda i: (0, 0)),
          pl.BlockSpec((U1, U2), lambda i: (0, 0)),
          pl.BlockSpec((1, U2), lambda i: (0, 0)),
      ],
      out_specs=pl.BlockSpec((TB, U2), lambda i: (i, 0)),
      out_shape=jax.ShapeDtypeStruct((B, U2), jnp.float32),
  )(x, b1, k2, b2)


def kernel(indices, values, kernel1, bias1, kernel2, bias2):
  idx = indices.astype(jnp.int32)
  table2 = kernel1.astype(jnp.bfloat16).reshape(500000, 128)
  parity = (idx & 1).astype(jnp.float32)
  vals = values.astype(jnp.float32)
  w0 = jnp.pad(vals * (1.0 - parity), ((0, 0), (0, 32 - F)))
  w1 = jnp.pad(vals * parity, ((0, 0), (0, 32 - F)))
  idx2 = (idx >> 1).reshape(NW, G, CH)
  out1 = _sc_embed(idx2, w0, w1, table2)
  perm = jnp.asarray(_PERM, dtype=jnp.int32)
  b1p = bias1[perm].reshape(1, U1)
  k2p = kernel2[perm, :]
  return _tc_tail(out1, b1p, k2p, bias2.reshape(1, U2))
